# baseline (speedup 1.0000x reference)
import functools

import jax
import jax.numpy as jnp
from jax import lax
from jax.experimental import pallas as pl
from jax.experimental.pallas import tpu as pltpu
from jax.experimental.pallas import tpu_sc as plsc

_NC = 2
_NS = 16
_NW = _NC * _NS
_B = 512
_L = 16


def _bcast_lane(v, l):
    idx = jnp.full((_L, 1), l, dtype=jnp.int32)
    return lax.gather(
        v, idx,
        lax.GatherDimensionNumbers(
            offset_dims=(), collapsed_slice_dims=(0,), start_index_map=(0,)),
        (1,),
        mode=lax.GatherScatterMode.PROMISE_IN_BOUNDS)


def _sc_scatter_mm(cols2d, vals2d, rows2d, matrix_a, zeros_nd, G, N, D):
    mesh = plsc.VectorSubcoreMesh(core_axis_name="c", subcore_axis_name="s")
    rows_per_tile = N // _NS

    @functools.partial(
        pl.kernel,
        mesh=mesh,
        out_type=jax.ShapeDtypeStruct((_NC, N, D), jnp.float32),
        scratch_types=[
            pltpu.VMEM((G, _B), jnp.int32),
            pltpu.VMEM((G, _B), jnp.float32),
            pltpu.VMEM((G, _B), jnp.int32),
            pltpu.VMEM((_B, D), jnp.float32),
            pltpu.VMEM_SHARED((N, D), jnp.float32),
            pltpu.SemaphoreType.DMA,
        ],
        compiler_params=pltpu.CompilerParams(use_tc_tiling_on_sc=False),
    )
    def k(cols_hbm, vals_hbm, rows_hbm, a_hbm, z_hbm, out_hbm,
          cols_v, vals_v, rows_v, gbuf, acc, gsem):
        c = lax.axis_index("c")
        s = lax.axis_index("s")
        wid = s * _NC + c

        pltpu.sync_copy(z_hbm.at[pl.ds(s * rows_per_tile, rows_per_tile)],
                        acc.at[pl.ds(s * rows_per_tile, rows_per_tile)])
        pltpu.sync_copy(cols_hbm.at[wid], cols_v)
        pltpu.sync_copy(vals_hbm.at[wid], vals_v)
        pltpu.sync_copy(rows_hbm.at[wid], rows_v)
        plsc.subcore_barrier()

        @pl.loop(0, G)
        def _(t):
            pltpu.async_copy(a_hbm.at[cols_v.at[t]], gbuf, gsem).wait()

            @pl.loop(0, _B // 128)
            def _(rb):
                base = rb * 128
                for j in range(128 // _L):
                    v16 = vals_v[t, pl.ds(base + j * _L, _L)]
                    for l in range(_L):
                        bv = _bcast_lane(v16, l)
                        r = base + j * _L + l
                        for d in range(D // _L):
                            sl = pl.ds(d * _L, _L)
                            gbuf[r, sl] = gbuf[r, sl] * bv

            pltpu.sync_copy(gbuf, acc.at[rows_v.at[t]], add=True)

        plsc.subcore_barrier()
        pltpu.sync_copy(acc.at[pl.ds(s * rows_per_tile, rows_per_tile)],
                        out_hbm.at[c, pl.ds(s * rows_per_tile, rows_per_tile)])

    return k(cols2d, vals2d, rows2d, matrix_a, zeros_nd)


def _combine(partials, N, D):
    blk = 1024

    def add_body(a_ref, b_ref, o_ref):
        o_ref[...] = a_ref[...] + b_ref[...]

    return pl.pallas_call(
        add_body,
        grid=(N // blk,),
        in_specs=[pl.BlockSpec((blk, D), lambda i: (i, 0)),
                  pl.BlockSpec((blk, D), lambda i: (i, 0))],
        out_specs=pl.BlockSpec((blk, D), lambda i: (i, 0)),
        out_shape=jax.ShapeDtypeStruct((N, D), jnp.float32),
    )(partials[0], partials[1])


def kernel(b_rows, b_cols, b_vals, matrix_A):
    nnz = b_rows.shape[0]
    N, D = matrix_A.shape
    per = _NW * _B
    G = -(-nnz // per)
    pad = G * per - nnz
    cols = jnp.concatenate(
        [b_cols.astype(jnp.int32), jnp.zeros((pad,), jnp.int32)]
    ).reshape(_NW, G, _B)
    vals = jnp.concatenate(
        [b_vals, jnp.zeros((pad,), jnp.float32)]).reshape(_NW, G, _B)
    rows = jnp.concatenate(
        [b_rows.astype(jnp.int32), jnp.zeros((pad,), jnp.int32)]
    ).reshape(_NW, G, _B)
    zeros_nd = jnp.zeros((N, D), jnp.float32)
    partials = _sc_scatter_mm(cols, vals, rows, matrix_A, zeros_nd, G, N, D)
    return _combine(partials, N, D)


# speedup vs baseline: 1.1290x; 1.1290x over previous
import functools

import jax
import jax.numpy as jnp
from jax import lax
from jax.experimental import pallas as pl
from jax.experimental.pallas import tpu as pltpu
from jax.experimental.pallas import tpu_sc as plsc

_NC = 2
_NS = 16
_NW = _NC * _NS
_B = 128
_L = 16


def _bcast_lane(v, l):
    idx = jnp.full((_L, 1), l, dtype=jnp.int32)
    return lax.gather(
        v, idx,
        lax.GatherDimensionNumbers(
            offset_dims=(), collapsed_slice_dims=(0,), start_index_map=(0,)),
        (1,),
        mode=lax.GatherScatterMode.PROMISE_IN_BOUNDS)


def _sc_scatter_mm(cols2d, vals2d, rows2d, matrix_a, zeros_nd, G, N, D):
    mesh = plsc.VectorSubcoreMesh(core_axis_name="c", subcore_axis_name="s")
    rows_per_tile = N // _NS

    NB = 4

    @functools.partial(
        pl.kernel,
        mesh=mesh,
        out_type=jax.ShapeDtypeStruct((_NC, N, D), jnp.float32),
        scratch_types=[
            pltpu.VMEM((G, _B), jnp.int32),
            pltpu.VMEM((G, _B), jnp.float32),
            pltpu.VMEM((G, _B), jnp.int32),
            pltpu.VMEM((NB, _B, D), jnp.float32),
            pltpu.VMEM_SHARED((N, D), jnp.float32),
            pltpu.SemaphoreType.DMA((NB,)),
            pltpu.SemaphoreType.DMA((NB,)),
        ],
        compiler_params=pltpu.CompilerParams(use_tc_tiling_on_sc=False),
    )
    def k(cols_hbm, vals_hbm, rows_hbm, a_hbm, z_hbm, out_hbm,
          cols_v, vals_v, rows_v, gbuf, acc, gsem, ssem):
        c = lax.axis_index("c")
        s = lax.axis_index("s")
        wid = s * _NC + c

        pltpu.sync_copy(z_hbm.at[pl.ds(s * rows_per_tile, rows_per_tile)],
                        acc.at[pl.ds(s * rows_per_tile, rows_per_tile)])
        pltpu.sync_copy(cols_hbm.at[wid], cols_v)
        pltpu.sync_copy(vals_hbm.at[wid], vals_v)
        pltpu.sync_copy(rows_hbm.at[wid], rows_v)
        plsc.subcore_barrier()

        def gather_start(t, b):
            pltpu.async_copy(a_hbm.at[cols_v.at[t]], gbuf.at[b], gsem.at[b])

        def gather_wait(t, b):
            pltpu.make_async_copy(
                a_hbm.at[cols_v.at[t]], gbuf.at[b], gsem.at[b]).wait()

        def scatter_start(t, b):
            pltpu.async_copy(
                gbuf.at[b], acc.at[rows_v.at[t]], ssem.at[b], add=True)

        def scatter_wait(t, b):
            pltpu.make_async_copy(
                gbuf.at[b], acc.at[rows_v.at[t]], ssem.at[b]).wait()

        def scale_group(t, b):
            gb = gbuf.at[b]

            @pl.loop(0, _B // _L)
            def _(j):
                v16 = vals_v[t, pl.ds(j * _L, _L)]
                for l in range(_L):
                    bv = _bcast_lane(v16, l)
                    r = j * _L + l
                    for d in range(D // _L):
                        sl = pl.ds(d * _L, _L)
                        gb[r, sl] = gb[r, sl] * bv

        def iteration(t, b, wait_prev_scatter=True, next_gather=True):
            gather_wait(t, b)
            scale_group(t, b)
            scatter_start(t, b)
            bn = (b + 2) % NB
            if wait_prev_scatter:
                scatter_wait(t - 2, bn)
            if next_gather:
                gather_start(t + 2, bn)

        gather_start(0, 0)
        gather_start(1, 1)
        iteration(0, 0, wait_prev_scatter=False)
        iteration(1, 1, wait_prev_scatter=False)
        iteration(2, 2)
        iteration(3, 3)

        @pl.loop(0, (G - 8) // NB)
        def _(u):
            t0 = 4 + u * NB
            for q in range(NB):
                iteration(t0 + q, q)

        iteration(G - 4, 0)
        iteration(G - 3, 1)
        iteration(G - 2, 2, next_gather=False)
        iteration(G - 1, 3, next_gather=False)
        scatter_wait(G - 2, 2)
        scatter_wait(G - 1, 3)
        plsc.subcore_barrier()
        pltpu.sync_copy(acc.at[pl.ds(s * rows_per_tile, rows_per_tile)],
                        out_hbm.at[c, pl.ds(s * rows_per_tile, rows_per_tile)])

    return k(cols2d, vals2d, rows2d, matrix_a, zeros_nd)


def _combine(partials, N, D):
    blk = 1024

    def add_body(a_ref, b_ref, o_ref):
        o_ref[...] = a_ref[...] + b_ref[...]

    return pl.pallas_call(
        add_body,
        grid=(N // blk,),
        in_specs=[pl.BlockSpec((blk, D), lambda i: (i, 0)),
                  pl.BlockSpec((blk, D), lambda i: (i, 0))],
        out_specs=pl.BlockSpec((blk, D), lambda i: (i, 0)),
        out_shape=jax.ShapeDtypeStruct((N, D), jnp.float32),
    )(partials[0], partials[1])


def kernel(b_rows, b_cols, b_vals, matrix_A):
    nnz = b_rows.shape[0]
    N, D = matrix_A.shape
    per = _NW * _B
    G = -(-nnz // per)
    G = max(-(-G // 4) * 4, 12)
    pad = G * per - nnz
    cols = jnp.concatenate(
        [b_cols.astype(jnp.int32), jnp.zeros((pad,), jnp.int32)]
    ).reshape(_NW, G, _B)
    vals = jnp.concatenate(
        [b_vals, jnp.zeros((pad,), jnp.float32)]).reshape(_NW, G, _B)
    rows = jnp.concatenate(
        [b_rows.astype(jnp.int32), jnp.zeros((pad,), jnp.int32)]
    ).reshape(_NW, G, _B)
    zeros_nd = jnp.zeros((N, D), jnp.float32)
    partials = _sc_scatter_mm(cols, vals, rows, matrix_A, zeros_nd, G, N, D)
    return _combine(partials, N, D)


# speedup vs baseline: 2.1761x; 1.9275x over previous
import functools

import jax
import jax.numpy as jnp
import numpy as np
from jax import lax
from jax.experimental import pallas as pl
from jax.experimental.pallas import tpu as pltpu
from jax.experimental.pallas import tpu_sc as plsc

_NC = 2
_NS = 16
_NW = _NC * _NS
_B = 128
_L = 16


def _col_perm(D):
    P = np.zeros(D, np.int32)
    for k in range(D // 32):
        for i in range(16):
            P[32 * k + 2 * i] = 32 * k + i
            P[32 * k + 2 * i + 1] = 32 * k + 16 + i
    return P


def _bcast_lane(v, l):
    idx = jnp.full((_L, 1), l, dtype=jnp.int32)
    return lax.gather(
        v, idx,
        lax.GatherDimensionNumbers(
            offset_dims=(), collapsed_slice_dims=(0,), start_index_map=(0,)),
        (1,),
        mode=lax.GatherScatterMode.PROMISE_IN_BOUNDS)


def _sc_scatter_mm(cols2d, vals2d, a_i32, zeros_nd, G, N, D):
    mesh = plsc.VectorSubcoreMesh(core_axis_name="c", subcore_axis_name="s")
    rows_per_tile = N // _NS
    W = D // 2

    @functools.partial(
        pl.kernel,
        mesh=mesh,
        out_type=jax.ShapeDtypeStruct((_NC, N, D), jnp.float32),
        scratch_types=[
            pltpu.VMEM((G, _B), jnp.int32),
            pltpu.VMEM((G, _B), jnp.float32),
            pltpu.VMEM((_B,), jnp.int32),
            pltpu.VMEM((_B,), jnp.int32),
            pltpu.VMEM((_B, W), jnp.int32),
            pltpu.VMEM((_B, D), jnp.float32),
            pltpu.VMEM_SHARED((N, W), jnp.int32),
            pltpu.VMEM_SHARED((N, D), jnp.float32),
            pltpu.SemaphoreType.DMA,
        ],
        compiler_params=pltpu.CompilerParams(use_tc_tiling_on_sc=False),
    )
    def k(pk_hbm, vals_hbm, ai_hbm, z_hbm, out_hbm,
          pk_v, vals_v, cols128, rows128, gbuf, fbuf, a_spm, acc, gsem):
        c = lax.axis_index("c")
        s = lax.axis_index("s")
        wid = s * _NC + c
        rsl = pl.ds(s * rows_per_tile, rows_per_tile)

        pltpu.sync_copy(ai_hbm.at[rsl], a_spm.at[rsl])
        pltpu.sync_copy(z_hbm.at[rsl], acc.at[rsl])
        pltpu.sync_copy(pk_hbm.at[wid], pk_v)
        pltpu.sync_copy(vals_hbm.at[wid], vals_v)
        plsc.subcore_barrier()

        hi_mask = jnp.full((_L,), -65536, dtype=jnp.int32)
        lo_mask = jnp.full((_L,), 0x3FFF, dtype=jnp.int32)

        @pl.loop(0, G)
        def _(t):
            for m in range(_B // _L):
                sl = pl.ds(m * _L, _L)
                w = pk_v[t, sl]
                cols128[sl] = jnp.bitwise_and(w, lo_mask)
                rows128[sl] = lax.shift_right_logical(
                    w, jnp.full((_L,), 14, jnp.int32))
            pltpu.async_copy(a_spm.at[cols128], gbuf, gsem).wait()
            for j in range(_B // _L):
                v16 = vals_v[t, pl.ds(j * _L, _L)]
                for l in range(_L):
                    bv = _bcast_lane(v16, l)
                    r = j * _L + l
                    for k2 in range(W // _L):
                        w = gbuf[r, pl.ds(k2 * _L, _L)]
                        ev = lax.bitcast_convert_type(
                            lax.shift_left(w, 16), jnp.float32)
                        od = lax.bitcast_convert_type(
                            jnp.bitwise_and(w, hi_mask), jnp.float32)
                        fbuf[r, pl.ds(32 * k2, _L)] = ev * bv
                        fbuf[r, pl.ds(32 * k2 + _L, _L)] = od * bv
            pltpu.sync_copy(fbuf, acc.at[rows128], add=True)

        plsc.subcore_barrier()
        pltpu.sync_copy(acc.at[rsl], out_hbm.at[c, rsl])

    return k(cols2d, vals2d, a_i32, zeros_nd)


def _combine(partials, N, D):
    blk = 1024

    def add_body(a_ref, b_ref, o_ref):
        o_ref[...] = a_ref[...] + b_ref[...]

    return pl.pallas_call(
        add_body,
        grid=(N // blk,),
        in_specs=[pl.BlockSpec((blk, D), lambda i: (i, 0)),
                  pl.BlockSpec((blk, D), lambda i: (i, 0))],
        out_specs=pl.BlockSpec((blk, D), lambda i: (i, 0)),
        out_shape=jax.ShapeDtypeStruct((N, D), jnp.float32),
    )(partials[0], partials[1])


def kernel(b_rows, b_cols, b_vals, matrix_A):
    nnz = b_rows.shape[0]
    N, D = matrix_A.shape
    per = _NW * _B
    G = -(-nnz // per)
    pad = G * per - nnz
    packed = jnp.concatenate(
        [(b_rows.astype(jnp.int32) << 14) | b_cols.astype(jnp.int32),
         jnp.zeros((pad,), jnp.int32)]).reshape(_NW, G, _B)
    vals = jnp.concatenate(
        [b_vals, jnp.zeros((pad,), jnp.float32)]).reshape(_NW, G, _B)
    a_perm = matrix_A[:, _col_perm(D)].astype(jnp.bfloat16)
    a_i32 = lax.bitcast_convert_type(
        a_perm.reshape(N, D // 2, 2), jnp.int32)
    zeros_nd = jnp.zeros((N, D), jnp.float32)
    partials = _sc_scatter_mm(packed, vals, a_i32, zeros_nd, G, N, D)
    return _combine(partials, N, D)


# speedup vs baseline: 2.4608x; 1.1308x over previous
import functools

import jax
import jax.numpy as jnp
import numpy as np
from jax import lax
from jax.experimental import pallas as pl
from jax.experimental.pallas import tpu as pltpu
from jax.experimental.pallas import tpu_sc as plsc

_NC = 2
_NS = 16
_NW = _NC * _NS
_B = 128
_L = 16


def _col_perm(D):
    P = np.zeros(D, np.int32)
    for k in range(D // 32):
        for i in range(16):
            P[32 * k + 2 * i] = 32 * k + i
            P[32 * k + 2 * i + 1] = 32 * k + 16 + i
    return P


def _bcast_lane(v, l):
    idx = jnp.full((_L, 1), l, dtype=jnp.int32)
    return lax.gather(
        v, idx,
        lax.GatherDimensionNumbers(
            offset_dims=(), collapsed_slice_dims=(0,), start_index_map=(0,)),
        (1,),
        mode=lax.GatherScatterMode.PROMISE_IN_BOUNDS)


def _sc_scatter_mm(pk1d, vals1d, a_i32, G, N, D):
    mesh = plsc.VectorSubcoreMesh(core_axis_name="c", subcore_axis_name="s")
    rows_per_tile = N // _NS
    W = D // 2
    GB = G * _B

    @functools.partial(
        pl.kernel,
        mesh=mesh,
        out_type=[jax.ShapeDtypeStruct((N, D), jnp.float32),
                  jax.ShapeDtypeStruct((N, D), jnp.float32)],
        scratch_types=[
            pltpu.VMEM((GB,), jnp.int32),
            pltpu.VMEM((GB,), jnp.float32),
            pltpu.VMEM((_B,), jnp.int32),
            pltpu.VMEM((_B,), jnp.int32),
            pltpu.VMEM((_B, W), jnp.int32),
            pltpu.VMEM((_B, D), jnp.float32),
            pltpu.VMEM_SHARED((N, W), jnp.int32),
            pltpu.VMEM_SHARED((N, D), jnp.float32),
            pltpu.SemaphoreType.DMA,
        ],
        compiler_params=pltpu.CompilerParams(use_tc_tiling_on_sc=False),
    )
    def k(pk_hbm, vals_hbm, ai_hbm, out0_hbm, out1_hbm,
          pk_v, vals_v, cols128, rows128, gbuf, fbuf, a_spm, acc, gsem):
        c = lax.axis_index("c")
        s = lax.axis_index("s")
        wid = s * _NC + c
        rsl = pl.ds(s * rows_per_tile, rows_per_tile)

        zero16 = jnp.zeros((_L,), jnp.float32)
        for r in range(_B):
            for d in range(D // _L):
                fbuf[r, pl.ds(d * _L, _L)] = zero16
        for z in range(rows_per_tile // _B):
            pltpu.sync_copy(
                fbuf, acc.at[pl.ds(s * rows_per_tile + z * _B, _B)])
        pltpu.sync_copy(ai_hbm.at[rsl], a_spm.at[rsl])
        pltpu.sync_copy(pk_hbm.at[pl.ds(wid * GB, GB)], pk_v)
        pltpu.sync_copy(vals_hbm.at[pl.ds(wid * GB, GB)], vals_v)
        plsc.subcore_barrier()

        hi_mask = jnp.full((_L,), -65536, dtype=jnp.int32)
        lo_mask = jnp.full((_L,), 0x3FFF, dtype=jnp.int32)

        @pl.loop(0, G)
        def _(t):
            for m in range(_B // _L):
                sl = pl.ds(m * _L, _L)
                w = pk_v[pl.ds(t * _B + m * _L, _L)]
                cols128[sl] = jnp.bitwise_and(w, lo_mask)
                rows128[sl] = lax.shift_right_logical(
                    w, jnp.full((_L,), 14, jnp.int32))
            pltpu.async_copy(a_spm.at[cols128], gbuf, gsem).wait()
            for j in range(_B // _L):
                v16 = vals_v[pl.ds(t * _B + j * _L, _L)]
                for l in range(_L):
                    bv = _bcast_lane(v16, l)
                    r = j * _L + l
                    for k2 in range(W // _L):
                        w = gbuf[r, pl.ds(k2 * _L, _L)]
                        ev = lax.bitcast_convert_type(
                            lax.shift_left(w, 16), jnp.float32)
                        od = lax.bitcast_convert_type(
                            jnp.bitwise_and(w, hi_mask), jnp.float32)
                        fbuf[r, pl.ds(32 * k2, _L)] = ev * bv
                        fbuf[r, pl.ds(32 * k2 + _L, _L)] = od * bv
            pltpu.sync_copy(fbuf, acc.at[rows128], add=True)

        plsc.subcore_barrier()
        @pl.when(c == 0)
        def _():
            pltpu.sync_copy(acc.at[rsl], out0_hbm.at[rsl])

        @pl.when(c == 1)
        def _():
            pltpu.sync_copy(acc.at[rsl], out1_hbm.at[rsl])

    return k(pk1d, vals1d, a_i32)


def _combine(p0, p1, N, D):
    blk = N // 4

    def add_body(a_ref, b_ref, o_ref):
        o_ref[...] = a_ref[...] + b_ref[...]

    return pl.pallas_call(
        add_body,
        grid=(N // blk,),
        in_specs=[pl.BlockSpec((blk, D), lambda i: (i, 0)),
                  pl.BlockSpec((blk, D), lambda i: (i, 0))],
        out_specs=pl.BlockSpec((blk, D), lambda i: (i, 0)),
        out_shape=jax.ShapeDtypeStruct((N, D), jnp.float32),
    )(p0, p1)


def kernel(b_rows, b_cols, b_vals, matrix_A):
    nnz = b_rows.shape[0]
    N, D = matrix_A.shape
    per = _NW * _B
    G = -(-nnz // per)
    pad = G * per - nnz
    packed = jnp.concatenate(
        [(b_rows.astype(jnp.int32) << 14) | b_cols.astype(jnp.int32),
         jnp.zeros((pad,), jnp.int32)])
    vals = jnp.concatenate([b_vals, jnp.zeros((pad,), jnp.float32)])
    a_perm = matrix_A[:, _col_perm(D)].astype(jnp.bfloat16)
    a_i32 = lax.bitcast_convert_type(
        a_perm.reshape(N, D // 2, 2), jnp.int32)
    p0, p1 = _sc_scatter_mm(packed, vals, a_i32, G, N, D)
    return _combine(p0, p1, N, D)


# speedup vs baseline: 2.7063x; 1.0998x over previous
import functools

import jax
import jax.numpy as jnp
import numpy as np
from jax import lax
from jax.experimental import pallas as pl
from jax.experimental.pallas import tpu as pltpu
from jax.experimental.pallas import tpu_sc as plsc

_NC = 2
_NS = 16
_NW = _NC * _NS
_B = 128
_L = 16


def _pack_a(matrix_a, N, D):
    a4 = matrix_a.reshape(N, D // 32, 2, 16).astype(jnp.bfloat16)
    lo = lax.bitcast_convert_type(a4[:, :, 0, :], jnp.uint16).astype(jnp.uint32)
    hi = lax.bitcast_convert_type(a4[:, :, 1, :], jnp.uint16).astype(jnp.uint32)
    w = (hi << 16) | lo
    return lax.bitcast_convert_type(w, jnp.int32).reshape(N, D // 2)


def _bcast_lane(v, l):
    idx = jnp.full((_L, 1), l, dtype=jnp.int32)
    return lax.gather(
        v, idx,
        lax.GatherDimensionNumbers(
            offset_dims=(), collapsed_slice_dims=(0,), start_index_map=(0,)),
        (1,),
        mode=lax.GatherScatterMode.PROMISE_IN_BOUNDS)


def _sc_scatter_mm(pk1d, vals1d, a_i32, G, N, D):
    mesh = plsc.VectorSubcoreMesh(core_axis_name="c", subcore_axis_name="s")
    rows_per_tile = N // _NS
    W = D // 2
    GB = G * _B

    @functools.partial(
        pl.kernel,
        mesh=mesh,
        out_type=[jax.ShapeDtypeStruct((N, D), jnp.float32),
                  jax.ShapeDtypeStruct((N, D), jnp.float32)],
        scratch_types=[
            pltpu.VMEM((GB,), jnp.int32),
            pltpu.VMEM((GB,), jnp.float32),
            pltpu.VMEM((_B,), jnp.int32),
            pltpu.VMEM((_B,), jnp.int32),
            pltpu.VMEM((_B, W), jnp.int32),
            pltpu.VMEM((_B, D), jnp.float32),
            pltpu.VMEM_SHARED((N, W), jnp.int32),
            pltpu.VMEM_SHARED((N, D), jnp.float32),
            pltpu.SemaphoreType.DMA,
        ],
        compiler_params=pltpu.CompilerParams(use_tc_tiling_on_sc=False),
    )
    def k(pk_hbm, vals_hbm, ai_hbm, out0_hbm, out1_hbm,
          pk_v, vals_v, cols128, rows128, gbuf, fbuf, a_spm, acc, gsem):
        c = lax.axis_index("c")
        s = lax.axis_index("s")
        wid = s * _NC + c
        rsl = pl.ds(s * rows_per_tile, rows_per_tile)

        zero16 = jnp.zeros((_L,), jnp.float32)
        for r in range(_B):
            for d in range(D // _L):
                fbuf[r, pl.ds(d * _L, _L)] = zero16
        for z in range(rows_per_tile // _B):
            pltpu.sync_copy(
                fbuf, acc.at[pl.ds(s * rows_per_tile + z * _B, _B)])
        pltpu.sync_copy(ai_hbm.at[rsl], a_spm.at[rsl])
        pltpu.sync_copy(pk_hbm.at[pl.ds(wid * GB, GB)], pk_v)
        pltpu.sync_copy(vals_hbm.at[pl.ds(wid * GB, GB)], vals_v)
        plsc.subcore_barrier()

        hi_mask = jnp.full((_L,), -65536, dtype=jnp.int32)
        lo_mask = jnp.full((_L,), 0x3FFF, dtype=jnp.int32)

        @pl.loop(0, G)
        def _(t):
            for m in range(_B // _L):
                sl = pl.ds(m * _L, _L)
                w = pk_v[pl.ds(t * _B + m * _L, _L)]
                cols128[sl] = jnp.bitwise_and(w, lo_mask)
                rows128[sl] = lax.shift_right_logical(
                    w, jnp.full((_L,), 14, jnp.int32))
            pltpu.async_copy(a_spm.at[cols128], gbuf, gsem).wait()
            for j in range(_B // _L):
                v16 = vals_v[pl.ds(t * _B + j * _L, _L)]
                for l in range(_L):
                    bv = _bcast_lane(v16, l)
                    r = j * _L + l
                    for k2 in range(W // _L):
                        w = gbuf[r, pl.ds(k2 * _L, _L)]
                        ev = lax.bitcast_convert_type(
                            lax.shift_left(w, 16), jnp.float32)
                        od = lax.bitcast_convert_type(
                            jnp.bitwise_and(w, hi_mask), jnp.float32)
                        fbuf[r, pl.ds(32 * k2, _L)] = ev * bv
                        fbuf[r, pl.ds(32 * k2 + _L, _L)] = od * bv
            pltpu.sync_copy(fbuf, acc.at[rows128], add=True)

        plsc.subcore_barrier()
        @pl.when(c == 0)
        def _():
            pltpu.sync_copy(acc.at[rsl], out0_hbm.at[rsl])

        @pl.when(c == 1)
        def _():
            pltpu.sync_copy(acc.at[rsl], out1_hbm.at[rsl])

    return k(pk1d, vals1d, a_i32)


def _combine(p0, p1, N, D):
    blk = N // 4

    def add_body(a_ref, b_ref, o_ref):
        o_ref[...] = a_ref[...] + b_ref[...]

    return pl.pallas_call(
        add_body,
        grid=(N // blk,),
        in_specs=[pl.BlockSpec((blk, D), lambda i: (i, 0)),
                  pl.BlockSpec((blk, D), lambda i: (i, 0))],
        out_specs=pl.BlockSpec((blk, D), lambda i: (i, 0)),
        out_shape=jax.ShapeDtypeStruct((N, D), jnp.float32),
    )(p0, p1)


def kernel(b_rows, b_cols, b_vals, matrix_A):
    nnz = b_rows.shape[0]
    N, D = matrix_A.shape
    per = _NW * _B
    G = -(-nnz // per)
    pad = G * per - nnz
    packed = jnp.concatenate(
        [(b_rows.astype(jnp.int32) << 14) | b_cols.astype(jnp.int32),
         jnp.zeros((pad,), jnp.int32)])
    vals = jnp.concatenate([b_vals, jnp.zeros((pad,), jnp.float32)])
    a_i32 = _pack_a(matrix_A, N, D)
    p0, p1 = _sc_scatter_mm(packed, vals, a_i32, G, N, D)
    return _combine(p0, p1, N, D)
